# Initial kernel scaffold; baseline (speedup 1.0000x reference)
import jax
import jax.numpy as jnp
from jax.experimental import pallas as pl


def kernel(x, W1, b1, W2, b2, gW, gb, tW, tb, nW1, nb1, nW2, nb2, bW1, bb1, bW2, bb2, edge_index, batch):
    raise NotImplementedError("write your pallas kernel here")


# speedup vs baseline: 32.6658x; 32.6658x over previous
import functools

import jax
import jax.numpy as jnp
from jax import lax
from jax.experimental import pallas as pl
from jax.experimental.pallas import tpu as pltpu
from jax.experimental.pallas import tpu_sc as plsc

N = 10000
E = 320000
G = 64

NC = 2
NS = 16
NW = NC * NS
CH = 128
CPT = 79
E_PAD = NW * CPT * CH
N_PAD = 10112
STRIP = N_PAD // NS

_sc_mesh = plsc.VectorSubcoreMesh(core_axis_name="c", subcore_axis_name="s")
_sc_params = pltpu.CompilerParams(use_tc_tiling_on_sc=False)


@functools.partial(
    pl.kernel,
    mesh=_sc_mesh,
    compiler_params=_sc_params,
    out_type=jax.ShapeDtypeStruct((NC, N_PAD), jnp.float32),
    scratch_types=[
        pltpu.VMEM((CPT, CH), jnp.int32),
        pltpu.VMEM((CH,), jnp.float32),
        pltpu.VMEM_SHARED((N_PAD,), jnp.float32),
        pltpu.SemaphoreType.DMA,
    ],
)
def _deg_count(dst2d, ones_hbm, zeros1d, out, dst_v, ones_v, deg_sh, sem):
    c = lax.axis_index("c")
    s = lax.axis_index("s")
    w = c * NS + s
    pltpu.sync_copy(zeros1d.at[pl.ds(s * STRIP, STRIP)],
                    deg_sh.at[pl.ds(s * STRIP, STRIP)])
    pltpu.sync_copy(dst2d.at[pl.ds(w * CPT, CPT)], dst_v)
    pltpu.sync_copy(ones_hbm, ones_v)
    plsc.subcore_barrier()

    def body(j, carry):
        pltpu.sync_copy(ones_v, deg_sh.at[dst_v.at[j]], add=True)
        return carry

    lax.fori_loop(0, CPT, body, 0)
    plsc.subcore_barrier()
    pltpu.sync_copy(deg_sh.at[pl.ds(s * STRIP, STRIP)],
                    out.at[c, pl.ds(s * STRIP, STRIP)])


@functools.partial(
    pl.kernel,
    mesh=_sc_mesh,
    compiler_params=_sc_params,
    out_type=jax.ShapeDtypeStruct((NC, N_PAD, 32), jnp.float32),
    scratch_types=[
        pltpu.VMEM((CPT, CH), jnp.int32),
        pltpu.VMEM((CPT, CH), jnp.int32),
        pltpu.VMEM((CH, 32), jnp.float32),
        pltpu.VMEM_SHARED((N_PAD, 32), jnp.float32),
        pltpu.SemaphoreType.DMA,
    ],
)
def _edge_agg(hs, src2d, dst2d, zeros2d, out, src_v, dst_v, rows_v, agg_sh, sem):
    c = lax.axis_index("c")
    s = lax.axis_index("s")
    w = c * NS + s
    pltpu.sync_copy(zeros2d.at[pl.ds(s * STRIP, STRIP)],
                    agg_sh.at[pl.ds(s * STRIP, STRIP)])
    pltpu.sync_copy(src2d.at[pl.ds(w * CPT, CPT)], src_v)
    pltpu.sync_copy(dst2d.at[pl.ds(w * CPT, CPT)], dst_v)
    plsc.subcore_barrier()

    def body(j, carry):
        pltpu.async_copy(hs.at[src_v.at[j]], rows_v, sem).wait()
        pltpu.sync_copy(rows_v, agg_sh.at[dst_v.at[j]], add=True)
        return carry

    lax.fori_loop(0, CPT, body, 0)
    plsc.subcore_barrier()
    pltpu.sync_copy(agg_sh.at[pl.ds(s * STRIP, STRIP)],
                    out.at[c, pl.ds(s * STRIP, STRIP)])


def _tc1_body(x_ref, w1_ref, d0_ref, d1_ref, hs1_ref):
    dis = lax.rsqrt(d0_ref[...] + d1_ref[...] + 1.0)
    h = jnp.dot(x_ref[...], w1_ref[...], preferred_element_type=jnp.float32)
    hs1_ref[...] = h * dis


def _tc2_body(hs1_ref, a0_ref, a1_ref, d0_ref, d1_ref, w2p_ref, b1_ref, hs2_ref):
    dis = lax.rsqrt(d0_ref[...] + d1_ref[...] + 1.0)
    h1 = jnp.maximum(dis * (a0_ref[...] + a1_ref[...] + hs1_ref[...]) + b1_ref[...], 0.0)
    h2 = jnp.dot(h1, w2p_ref[...], preferred_element_type=jnp.float32)
    hs2_ref[...] = h2 * dis


def _tc3_body(hs2_ref, a0_ref, a1_ref, d0_ref, d1_ref, b2p_ref,
              brow_ref, bcol_ref,
              gwp_ref, gb_ref, tw_ref, tb_ref,
              nw1p_ref, nb1_ref, nw2_ref, nb2_ref,
              bw1p_ref, bb1_ref, bw2_ref, bb2_ref,
              t_ref, nsoft_ref, bout_ref):
    f32 = jnp.float32
    dis = lax.rsqrt(d0_ref[...] + d1_ref[...] + 1.0)
    h2 = jnp.maximum(dis * (a0_ref[...] + a1_ref[...] + hs2_ref[...]) + b2p_ref[...], 0.0)
    gi_row = lax.broadcasted_iota(jnp.int32, (G, N), 0)
    gi_col = lax.broadcasted_iota(jnp.int32, (N, G), 1)
    M = (gi_row == brow_ref[...]).astype(f32)
    MT = (gi_col == bcol_ref[...]).astype(f32)
    cnt = jnp.sum(M, axis=1, keepdims=True)
    seg = jnp.dot(M, h2, preferred_element_type=f32)
    gmean = seg / jnp.maximum(cnt, 1.0)
    g1 = jnp.dot(gmean, gwp_ref[...], preferred_element_type=f32) + gb_ref[...]
    tl = jnp.dot(g1, tw_ref[...], preferred_element_type=f32) + tb_ref[...]
    tl = tl - jnp.max(tl, axis=1, keepdims=True)
    te = jnp.exp(tl)
    t_ref[...] = te / jnp.sum(te, axis=1, keepdims=True)
    nh = jnp.maximum(jnp.dot(h2, nw1p_ref[...], preferred_element_type=f32) + nb1_ref[...], 0.0)
    n2 = jnp.dot(nh, nw2_ref[...], preferred_element_type=f32) + nb2_ref[...]
    mu = jnp.dot(M, n2, preferred_element_type=f32) / jnp.maximum(cnt, 1.0)
    mcol = jnp.dot(MT, mu, preferred_element_type=f32)
    nexp = jnp.exp(n2 - mcol)
    den = jnp.dot(M, nexp, preferred_element_type=f32)
    dencol = jnp.dot(MT, den, preferred_element_type=f32)
    nsoft = nexp / dencol
    nsoft_ref[...] = nsoft
    bpool = jnp.dot(M, nsoft * h2, preferred_element_type=f32)
    bh = jnp.maximum(jnp.dot(bpool, bw1p_ref[...], preferred_element_type=f32) + bb1_ref[...], 0.0)
    bl = jnp.dot(bh, bw2_ref[...], preferred_element_type=f32) + bb2_ref[...]
    bl = bl - jnp.max(bl, axis=0, keepdims=True)
    be = jnp.exp(bl)
    bout_ref[...] = be / jnp.sum(be, axis=0, keepdims=True)


def _pad_rows(w, rows):
    return jnp.zeros((rows, w.shape[1]), jnp.float32).at[: w.shape[0], :].set(w)


def kernel(x, W1, b1, W2, b2, gW, gb, tW, tb, nW1, nb1, nW2, nb2, bW1, bb1, bW2, bb2, edge_index, batch):
    src = edge_index[0]
    dst = edge_index[1]
    pad_n = E_PAD - E
    pad_src = (lax.iota(jnp.int32, pad_n) * 131) % N
    pad_dst = N + (lax.iota(jnp.int32, pad_n) % (N_PAD - N))
    src2d = jnp.concatenate([src, pad_src]).reshape(NW * CPT, CH)
    dst2d = jnp.concatenate([dst, pad_dst]).reshape(NW * CPT, CH)
    zeros1d = jnp.zeros((N_PAD,), jnp.float32)
    zeros2d = jnp.zeros((N_PAD, 32), jnp.float32)
    ones_ch = jnp.ones((CH,), jnp.float32)

    degp = _deg_count(dst2d, ones_ch, zeros1d)
    d0 = degp[0, :N, None]
    d1 = degp[1, :N, None]

    hs1 = pl.pallas_call(
        _tc1_body,
        out_shape=jax.ShapeDtypeStruct((N, 32), jnp.float32),
    )(x, W1, d0, d1)

    aggp1 = _edge_agg(hs1, src2d, dst2d, zeros2d)

    w2p = jnp.zeros((32, 32), jnp.float32).at[:, :24].set(W2)
    hs2 = pl.pallas_call(
        _tc2_body,
        out_shape=jax.ShapeDtypeStruct((N, 32), jnp.float32),
    )(hs1, aggp1[0, :N], aggp1[1, :N], d0, d1, w2p, b1[None, :])

    aggp2 = _edge_agg(hs2, src2d, dst2d, zeros2d)

    b2p = jnp.zeros((1, 32), jnp.float32).at[0, :24].set(b2)
    gwp = _pad_rows(gW, 32)
    nw1p = _pad_rows(nW1, 32)
    bw1p = _pad_rows(bW1, 32)
    brow = batch[None, :].astype(jnp.int32)
    bcol = batch[:, None].astype(jnp.int32)
    t, n_soft, b_out = pl.pallas_call(
        _tc3_body,
        out_shape=[
            jax.ShapeDtypeStruct((G, 2), jnp.float32),
            jax.ShapeDtypeStruct((N, 1), jnp.float32),
            jax.ShapeDtypeStruct((G, 3), jnp.float32),
        ],
    )(hs2, aggp2[0, :N], aggp2[1, :N], d0, d1, b2p,
      brow, bcol,
      gwp, gb[None, :], tW, tb[None, :],
      nw1p, nb1[None, :], nW2, nb2[None, :],
      bw1p, bb1[None, :], bW2, bb2[None, :])
    return (t, n_soft, b_out)


# speedup vs baseline: 49.5035x; 1.5155x over previous
import functools

import jax
import jax.numpy as jnp
from jax import lax
from jax.experimental import pallas as pl
from jax.experimental.pallas import tpu as pltpu
from jax.experimental.pallas import tpu_sc as plsc

N = 10000
E = 320000
G = 64

NC = 2
NS = 16
NW = NC * NS
CH = 128
CPT = 80
NBUF = 4
E_PAD = NW * CPT * CH
N_PAD = 10112
STRIP = N_PAD // NS

_sc_mesh = plsc.VectorSubcoreMesh(core_axis_name="c", subcore_axis_name="s")
_sc_params = pltpu.CompilerParams(use_tc_tiling_on_sc=False)


@functools.partial(
    pl.kernel,
    mesh=_sc_mesh,
    compiler_params=_sc_params,
    out_type=jax.ShapeDtypeStruct((NC, N_PAD), jnp.float32),
    scratch_types=[
        pltpu.VMEM((CPT, CH), jnp.int32),
        pltpu.VMEM((CH,), jnp.float32),
        pltpu.VMEM_SHARED((N_PAD,), jnp.float32),
        pltpu.SemaphoreType.DMA,
    ],
)
def _deg_count(dst2d, ones_hbm, zeros1d, out, dst_v, ones_v, deg_sh, sem):
    c = lax.axis_index("c")
    s = lax.axis_index("s")
    w = c * NS + s
    pltpu.sync_copy(zeros1d.at[pl.ds(s * STRIP, STRIP)],
                    deg_sh.at[pl.ds(s * STRIP, STRIP)])
    pltpu.sync_copy(dst2d.at[pl.ds(w * CPT, CPT)], dst_v)
    pltpu.sync_copy(ones_hbm, ones_v)
    plsc.subcore_barrier()

    def body(j, carry):
        pltpu.sync_copy(ones_v, deg_sh.at[dst_v.at[j]], add=True)
        return carry

    lax.fori_loop(0, CPT, body, 0)
    plsc.subcore_barrier()
    pltpu.sync_copy(deg_sh.at[pl.ds(s * STRIP, STRIP)],
                    out.at[c, pl.ds(s * STRIP, STRIP)])


@functools.partial(
    pl.kernel,
    mesh=_sc_mesh,
    compiler_params=_sc_params,
    out_type=jax.ShapeDtypeStruct((NC, N_PAD, 32), jnp.float32),
    scratch_types=[
        pltpu.VMEM((CPT, CH), jnp.int32),
        pltpu.VMEM((CPT, CH), jnp.int32),
        pltpu.VMEM((NBUF, CH, 32), jnp.float32),
        pltpu.VMEM_SHARED((N_PAD, 32), jnp.float32),
        pltpu.SemaphoreType.DMA((NBUF,)),
    ],
)
def _edge_agg(hs, src2d, dst2d, zeros2d, out, src_v, dst_v, rows_v, agg_sh, sem):
    c = lax.axis_index("c")
    s = lax.axis_index("s")
    w = c * NS + s
    pltpu.sync_copy(zeros2d.at[pl.ds(s * STRIP, STRIP)],
                    agg_sh.at[pl.ds(s * STRIP, STRIP)])
    pltpu.sync_copy(src2d.at[pl.ds(w * CPT, CPT)], src_v)
    pltpu.sync_copy(dst2d.at[pl.ds(w * CPT, CPT)], dst_v)
    plsc.subcore_barrier()

    for b in range(NBUF):
        pltpu.async_copy(hs.at[src_v.at[b]], rows_v.at[b], sem.at[b])

    def outer(g, carry):
        for b in range(NBUF):
            j = g * NBUF + b
            pltpu.make_async_copy(hs.at[src_v.at[j]], rows_v.at[b], sem.at[b]).wait()
            pltpu.sync_copy(rows_v.at[b], agg_sh.at[dst_v.at[j]], add=True)
            pltpu.async_copy(hs.at[src_v.at[j + NBUF]], rows_v.at[b], sem.at[b])
        return carry

    lax.fori_loop(0, (CPT - NBUF) // NBUF, outer, 0)
    for b in range(NBUF):
        j = CPT - NBUF + b
        pltpu.make_async_copy(hs.at[src_v.at[j]], rows_v.at[b], sem.at[b]).wait()
        pltpu.sync_copy(rows_v.at[b], agg_sh.at[dst_v.at[j]], add=True)
    plsc.subcore_barrier()
    pltpu.sync_copy(agg_sh.at[pl.ds(s * STRIP, STRIP)],
                    out.at[c, pl.ds(s * STRIP, STRIP)])


def _tc1_body(x_ref, w1_ref, d0_ref, d1_ref, hs1_ref):
    dis = lax.rsqrt(d0_ref[...] + d1_ref[...] + 1.0)
    h = jnp.dot(x_ref[...], w1_ref[...], preferred_element_type=jnp.float32)
    hs1_ref[...] = h * dis


def _tc2_body(hs1_ref, a0_ref, a1_ref, d0_ref, d1_ref, w2p_ref, b1_ref, hs2_ref):
    dis = lax.rsqrt(d0_ref[...] + d1_ref[...] + 1.0)
    h1 = jnp.maximum(dis * (a0_ref[...] + a1_ref[...] + hs1_ref[...]) + b1_ref[...], 0.0)
    h2 = jnp.dot(h1, w2p_ref[...], preferred_element_type=jnp.float32)
    hs2_ref[...] = h2 * dis


def _tc3_body(hs2_ref, a0_ref, a1_ref, d0_ref, d1_ref, b2p_ref,
              brow_ref, bcol_ref,
              gwp_ref, gb_ref, tw_ref, tb_ref,
              nw1p_ref, nb1_ref, nw2_ref, nb2_ref,
              bw1p_ref, bb1_ref, bw2_ref, bb2_ref,
              t_ref, nsoft_ref, bout_ref):
    f32 = jnp.float32
    dis = lax.rsqrt(d0_ref[...] + d1_ref[...] + 1.0)
    h2 = jnp.maximum(dis * (a0_ref[...] + a1_ref[...] + hs2_ref[...]) + b2p_ref[...], 0.0)
    gi_row = lax.broadcasted_iota(jnp.int32, (G, N), 0)
    gi_col = lax.broadcasted_iota(jnp.int32, (N, G), 1)
    M = (gi_row == brow_ref[...]).astype(f32)
    MT = (gi_col == bcol_ref[...]).astype(f32)
    cnt = jnp.sum(M, axis=1, keepdims=True)
    seg = jnp.dot(M, h2, preferred_element_type=f32)
    gmean = seg / jnp.maximum(cnt, 1.0)
    g1 = jnp.dot(gmean, gwp_ref[...], preferred_element_type=f32) + gb_ref[...]
    tl = jnp.dot(g1, tw_ref[...], preferred_element_type=f32) + tb_ref[...]
    tl = tl - jnp.max(tl, axis=1, keepdims=True)
    te = jnp.exp(tl)
    t_ref[...] = te / jnp.sum(te, axis=1, keepdims=True)
    nh = jnp.maximum(jnp.dot(h2, nw1p_ref[...], preferred_element_type=f32) + nb1_ref[...], 0.0)
    n2 = jnp.dot(nh, nw2_ref[...], preferred_element_type=f32) + nb2_ref[...]
    mu = jnp.dot(M, n2, preferred_element_type=f32) / jnp.maximum(cnt, 1.0)
    mcol = jnp.dot(MT, mu, preferred_element_type=f32)
    nexp = jnp.exp(n2 - mcol)
    den = jnp.dot(M, nexp, preferred_element_type=f32)
    dencol = jnp.dot(MT, den, preferred_element_type=f32)
    nsoft = nexp / dencol
    nsoft_ref[...] = nsoft
    bpool = jnp.dot(M, nsoft * h2, preferred_element_type=f32)
    bh = jnp.maximum(jnp.dot(bpool, bw1p_ref[...], preferred_element_type=f32) + bb1_ref[...], 0.0)
    bl = jnp.dot(bh, bw2_ref[...], preferred_element_type=f32) + bb2_ref[...]
    bl = bl - jnp.max(bl, axis=0, keepdims=True)
    be = jnp.exp(bl)
    bout_ref[...] = be / jnp.sum(be, axis=0, keepdims=True)


def _pad_rows(w, rows):
    return jnp.zeros((rows, w.shape[1]), jnp.float32).at[: w.shape[0], :].set(w)


def kernel(x, W1, b1, W2, b2, gW, gb, tW, tb, nW1, nb1, nW2, nb2, bW1, bb1, bW2, bb2, edge_index, batch):
    src = edge_index[0]
    dst = edge_index[1]
    pad_n = E_PAD - E
    pad_src = (lax.iota(jnp.int32, pad_n) * 131) % N
    pad_dst = N + (lax.iota(jnp.int32, pad_n) % (N_PAD - N))
    src2d = jnp.concatenate([src, pad_src]).reshape(NW * CPT, CH)
    dst2d = jnp.concatenate([dst, pad_dst]).reshape(NW * CPT, CH)
    zeros1d = jnp.zeros((N_PAD,), jnp.float32)
    zeros2d = jnp.zeros((N_PAD, 32), jnp.float32)
    ones_ch = jnp.ones((CH,), jnp.float32)

    degp = _deg_count(dst2d, ones_ch, zeros1d)
    d0 = degp[0, :N, None]
    d1 = degp[1, :N, None]

    hs1 = pl.pallas_call(
        _tc1_body,
        out_shape=jax.ShapeDtypeStruct((N, 32), jnp.float32),
    )(x, W1, d0, d1)

    aggp1 = _edge_agg(hs1, src2d, dst2d, zeros2d)

    w2p = jnp.zeros((32, 32), jnp.float32).at[:, :24].set(W2)
    hs2 = pl.pallas_call(
        _tc2_body,
        out_shape=jax.ShapeDtypeStruct((N, 32), jnp.float32),
    )(hs1, aggp1[0, :N], aggp1[1, :N], d0, d1, w2p, b1[None, :])

    aggp2 = _edge_agg(hs2, src2d, dst2d, zeros2d)

    b2p = jnp.zeros((1, 32), jnp.float32).at[0, :24].set(b2)
    gwp = _pad_rows(gW, 32)
    nw1p = _pad_rows(nW1, 32)
    bw1p = _pad_rows(bW1, 32)
    brow = batch[None, :].astype(jnp.int32)
    bcol = batch[:, None].astype(jnp.int32)
    t, n_soft, b_out = pl.pallas_call(
        _tc3_body,
        out_shape=[
            jax.ShapeDtypeStruct((G, 2), jnp.float32),
            jax.ShapeDtypeStruct((N, 1), jnp.float32),
            jax.ShapeDtypeStruct((G, 3), jnp.float32),
        ],
    )(hs2, aggp2[0, :N], aggp2[1, :N], d0, d1, b2p,
      brow, bcol,
      gwp, gb[None, :], tW, tb[None, :],
      nw1p, nb1[None, :], nW2, nb2[None, :],
      bw1p, bb1[None, :], bW2, bb2[None, :])
    return (t, n_soft, b_out)


# speedup vs baseline: 54.9202x; 1.1094x over previous
import functools

import jax
import jax.numpy as jnp
from jax import lax
from jax.experimental import pallas as pl
from jax.experimental.pallas import tpu as pltpu
from jax.experimental.pallas import tpu_sc as plsc

N = 10000
E = 320000
G = 64

NC = 2
NS = 16
NW = NC * NS
CH = 125
CPT = 80
NBUF = 4
N_PAD = 10112
STRIP = N_PAD // NS

_sc_mesh = plsc.VectorSubcoreMesh(core_axis_name="c", subcore_axis_name="s")
_sc_params = pltpu.CompilerParams(use_tc_tiling_on_sc=False)


@functools.partial(
    pl.kernel,
    mesh=_sc_mesh,
    compiler_params=_sc_params,
    out_type=jax.ShapeDtypeStruct((NC, N_PAD), jnp.float32),
    scratch_types=[
        pltpu.VMEM((CPT, CH), jnp.int32),
        pltpu.VMEM((CH,), jnp.float32),
        pltpu.VMEM_SHARED((N_PAD,), jnp.float32),
        pltpu.SemaphoreType.DMA,
    ],
)
def _deg_count(dst2d, ones_hbm, zeros1d, out, dst_v, ones_v, deg_sh, sem):
    c = lax.axis_index("c")
    s = lax.axis_index("s")
    w = c * NS + s
    pltpu.sync_copy(zeros1d.at[pl.ds(s * STRIP, STRIP)],
                    deg_sh.at[pl.ds(s * STRIP, STRIP)])
    pltpu.sync_copy(dst2d.at[pl.ds(w * CPT, CPT)], dst_v)
    pltpu.sync_copy(ones_hbm, ones_v)
    plsc.subcore_barrier()

    def body(j, carry):
        pltpu.sync_copy(ones_v, deg_sh.at[dst_v.at[j]], add=True)
        return carry

    lax.fori_loop(0, CPT, body, 0)
    plsc.subcore_barrier()
    pltpu.sync_copy(deg_sh.at[pl.ds(s * STRIP, STRIP)],
                    out.at[c, pl.ds(s * STRIP, STRIP)])


@functools.partial(
    pl.kernel,
    mesh=_sc_mesh,
    compiler_params=_sc_params,
    out_type=jax.ShapeDtypeStruct((NC, N_PAD, 32), jnp.float32),
    scratch_types=[
        pltpu.VMEM((CPT, CH), jnp.int32),
        pltpu.VMEM((CPT, CH), jnp.int32),
        pltpu.VMEM((NBUF, CH, 32), jnp.float32),
        pltpu.VMEM_SHARED((N_PAD, 32), jnp.float32),
        pltpu.SemaphoreType.DMA((NBUF,)),
    ],
)
def _edge_agg(hs, src2d, dst2d, zeros2d, out, src_v, dst_v, rows_v, agg_sh, sem):
    c = lax.axis_index("c")
    s = lax.axis_index("s")
    w = c * NS + s
    pltpu.sync_copy(zeros2d.at[pl.ds(s * STRIP, STRIP)],
                    agg_sh.at[pl.ds(s * STRIP, STRIP)])
    pltpu.sync_copy(src2d.at[pl.ds(w * CPT, CPT)], src_v)
    pltpu.sync_copy(dst2d.at[pl.ds(w * CPT, CPT)], dst_v)
    plsc.subcore_barrier()

    for b in range(NBUF):
        pltpu.async_copy(hs.at[src_v.at[b]], rows_v.at[b], sem.at[b])

    def outer(g, carry):
        for b in range(NBUF):
            j = g * NBUF + b
            pltpu.make_async_copy(hs.at[src_v.at[j]], rows_v.at[b], sem.at[b]).wait()
            pltpu.sync_copy(rows_v.at[b], agg_sh.at[dst_v.at[j]], add=True)
            pltpu.async_copy(hs.at[src_v.at[j + NBUF]], rows_v.at[b], sem.at[b])
        return carry

    lax.fori_loop(0, (CPT - NBUF) // NBUF, outer, 0)
    for b in range(NBUF):
        j = CPT - NBUF + b
        pltpu.make_async_copy(hs.at[src_v.at[j]], rows_v.at[b], sem.at[b]).wait()
        pltpu.sync_copy(rows_v.at[b], agg_sh.at[dst_v.at[j]], add=True)
    plsc.subcore_barrier()
    pltpu.sync_copy(agg_sh.at[pl.ds(s * STRIP, STRIP)],
                    out.at[c, pl.ds(s * STRIP, STRIP)])


def _tc1_body(x_ref, w1_ref, d0_ref, d1_ref, hs1_ref):
    dis = lax.rsqrt(d0_ref[...] + d1_ref[...] + 1.0)
    h = jnp.dot(x_ref[...], w1_ref[...], preferred_element_type=jnp.float32)
    hs1_ref[...] = h * dis


def _tc2_body(hs1_ref, agg_ref, d0_ref, d1_ref, w2p_ref, b1_ref, hs2_ref):
    dis = lax.rsqrt(d0_ref[...] + d1_ref[...] + 1.0)
    a0 = agg_ref[0, :N, :]
    a1 = agg_ref[1, :N, :]
    h1 = jnp.maximum(dis * (a0 + a1 + hs1_ref[...]) + b1_ref[...], 0.0)
    h2 = jnp.dot(h1, w2p_ref[...], preferred_element_type=jnp.float32)
    hs2_ref[...] = h2 * dis


def _tc3_body(hs2_ref, agg_ref, d0_ref, d1_ref, b2p_ref,
              brow_ref, bcol_ref,
              gwp_ref, gb_ref, tw_ref, tb_ref,
              nw1p_ref, nb1_ref, nw2_ref, nb2_ref,
              bw1p_ref, bb1_ref, bw2_ref, bb2_ref,
              t_ref, nsoft_ref, bout_ref):
    f32 = jnp.float32
    dis = lax.rsqrt(d0_ref[...] + d1_ref[...] + 1.0)
    a0 = agg_ref[0, :N, :]
    a1 = agg_ref[1, :N, :]
    h2 = jnp.maximum(dis * (a0 + a1 + hs2_ref[...]) + b2p_ref[...], 0.0)
    gi_row = lax.broadcasted_iota(jnp.int32, (G, N), 0)
    gi_col = lax.broadcasted_iota(jnp.int32, (N, G), 1)
    M = (gi_row == brow_ref[...]).astype(f32)
    MT = (gi_col == bcol_ref[...]).astype(f32)
    cnt = jnp.sum(M, axis=1, keepdims=True)
    seg = jnp.dot(M, h2, preferred_element_type=f32)
    gmean = seg / jnp.maximum(cnt, 1.0)
    g1 = jnp.dot(gmean, gwp_ref[...], preferred_element_type=f32) + gb_ref[...]
    tl = jnp.dot(g1, tw_ref[...], preferred_element_type=f32) + tb_ref[...]
    tl = tl - jnp.max(tl, axis=1, keepdims=True)
    te = jnp.exp(tl)
    t_ref[...] = te / jnp.sum(te, axis=1, keepdims=True)
    nh = jnp.maximum(jnp.dot(h2, nw1p_ref[...], preferred_element_type=f32) + nb1_ref[...], 0.0)
    n2 = jnp.dot(nh, nw2_ref[...], preferred_element_type=f32) + nb2_ref[...]
    mu = jnp.dot(M, n2, preferred_element_type=f32) / jnp.maximum(cnt, 1.0)
    mcol = jnp.dot(MT, mu, preferred_element_type=f32)
    nexp = jnp.exp(n2 - mcol)
    den = jnp.dot(M, nexp, preferred_element_type=f32)
    dencol = jnp.dot(MT, den, preferred_element_type=f32)
    nsoft = nexp / dencol
    nsoft_ref[...] = nsoft
    bpool = jnp.dot(M, nsoft * h2, preferred_element_type=f32)
    bh = jnp.maximum(jnp.dot(bpool, bw1p_ref[...], preferred_element_type=f32) + bb1_ref[...], 0.0)
    bl = jnp.dot(bh, bw2_ref[...], preferred_element_type=f32) + bb2_ref[...]
    bl = bl - jnp.max(bl, axis=0, keepdims=True)
    be = jnp.exp(bl)
    bout_ref[...] = be / jnp.sum(be, axis=0, keepdims=True)


def _pad_rows(w, rows):
    return jnp.zeros((rows, w.shape[1]), jnp.float32).at[: w.shape[0], :].set(w)


def kernel(x, W1, b1, W2, b2, gW, gb, tW, tb, nW1, nb1, nW2, nb2, bW1, bb1, bW2, bb2, edge_index, batch):
    src2d = edge_index[0].reshape(NW * CPT, CH)
    dst2d = edge_index[1].reshape(NW * CPT, CH)
    zeros1d = jnp.zeros((N_PAD,), jnp.float32)
    zeros2d = jnp.zeros((N_PAD, 32), jnp.float32)
    ones_ch = jnp.ones((CH,), jnp.float32)

    degp = _deg_count(dst2d, ones_ch, zeros1d)
    d0 = degp[0, :N, None]
    d1 = degp[1, :N, None]

    hs1 = pl.pallas_call(
        _tc1_body,
        out_shape=jax.ShapeDtypeStruct((N, 32), jnp.float32),
    )(x, W1, d0, d1)

    aggp1 = _edge_agg(hs1, src2d, dst2d, zeros2d)

    w2p = jnp.zeros((32, 32), jnp.float32).at[:, :24].set(W2)
    hs2 = pl.pallas_call(
        _tc2_body,
        out_shape=jax.ShapeDtypeStruct((N, 32), jnp.float32),
    )(hs1, aggp1, d0, d1, w2p, b1[None, :])

    aggp2 = _edge_agg(hs2, src2d, dst2d, zeros2d)

    b2p = jnp.zeros((1, 32), jnp.float32).at[0, :24].set(b2)
    gwp = _pad_rows(gW, 32)
    nw1p = _pad_rows(nW1, 32)
    bw1p = _pad_rows(bW1, 32)
    brow = batch[None, :].astype(jnp.int32)
    bcol = batch[:, None].astype(jnp.int32)
    t, n_soft, b_out = pl.pallas_call(
        _tc3_body,
        out_shape=[
            jax.ShapeDtypeStruct((G, 2), jnp.float32),
            jax.ShapeDtypeStruct((N, 1), jnp.float32),
            jax.ShapeDtypeStruct((G, 3), jnp.float32),
        ],
    )(hs2, aggp2, d0, d1, b2p,
      brow, bcol,
      gwp, gb[None, :], tW, tb[None, :],
      nw1p, nb1[None, :], nW2, nb2[None, :],
      bw1p, bb1[None, :], bW2, bb2[None, :])
    return (t, n_soft, b_out)
